# Initial kernel scaffold; baseline (speedup 1.0000x reference)
import jax
import jax.numpy as jnp
from jax.experimental import pallas as pl


def kernel(x, edge_index, W):
    raise NotImplementedError("write your pallas kernel here")


# speedup vs baseline: 5.1035x; 5.1035x over previous
import functools

import jax
import jax.numpy as jnp
from jax import lax
from jax.experimental import pallas as pl
from jax.experimental.pallas import tpu as pltpu
from jax.experimental.pallas import tpu_sc as plsc

N_NODES = 10000
N_EDGES = 320000
D = 128

NC = 2
NS = 16
NW = NC * NS
EPT = N_EDGES // NW
K = 80
NCHUNK = EPT // K
N_PAD = 10240
RPT = N_PAD // NS
RSTG = 128
NSTG = RPT // RSTG


def _sc_body(x_hbm, src_hbm, dst_hbm, out_hbm,
             src_v, dst_v, rows_v, stg_v, acc, gsem):
    cid = lax.axis_index("c")
    sid = lax.axis_index("s")
    wid = sid * NC + cid

    zeros16 = jnp.zeros((16,), jnp.float32)

    def zrow(i, c):
        for j in range(D // 16):
            stg_v[i, pl.ds(j * 16, 16)] = zeros16
        return c

    lax.fori_loop(0, RSTG, zrow, 0)
    for t in range(NSTG):
        pltpu.sync_copy(stg_v, acc.at[pl.ds(sid * RPT + t * RSTG, RSTG)])
    plsc.subcore_barrier()

    def ebody(j, c):
        base = wid * EPT + j * K
        pltpu.sync_copy(src_hbm.at[pl.ds(base, K)], src_v)
        pltpu.sync_copy(dst_hbm.at[pl.ds(base, K)], dst_v)
        pltpu.async_copy(x_hbm.at[src_v], rows_v, gsem).wait()
        pltpu.sync_copy(rows_v, acc.at[dst_v], add=True)
        return c

    lax.fori_loop(0, NCHUNK, ebody, 0)
    plsc.subcore_barrier()

    for t in range(NSTG):
        r0 = sid * RPT + t * RSTG
        pltpu.sync_copy(acc.at[pl.ds(r0, RSTG)], stg_v)
        pltpu.sync_copy(stg_v, out_hbm.at[cid, pl.ds(r0, RSTG)])


_sc_scatter = functools.partial(
    pl.kernel,
    out_type=jax.ShapeDtypeStruct((NC, N_PAD, D), jnp.float32),
    mesh=plsc.VectorSubcoreMesh(core_axis_name="c", subcore_axis_name="s"),
    scratch_types=[
        pltpu.VMEM((K,), jnp.int32),
        pltpu.VMEM((K,), jnp.int32),
        pltpu.VMEM((K, D), jnp.float32),
        pltpu.VMEM((RSTG, D), jnp.float32),
        pltpu.VMEM_SHARED((N_PAD, D), jnp.float32),
        pltpu.SemaphoreType.DMA,
    ],
)(_sc_body)


ROWS_BLK = 1000


def _tc_body(p_ref, w_ref, o_ref):
    s = p_ref[0] + p_ref[1]
    o_ref[...] = jnp.maximum(
        lax.dot_general(s, w_ref[...], (((1,), (1,)), ((), ())),
                        preferred_element_type=jnp.float32),
        0.0)


def _combine(partials, W):
    return pl.pallas_call(
        _tc_body,
        grid=(N_NODES // ROWS_BLK,),
        in_specs=[
            pl.BlockSpec((NC, ROWS_BLK, D), lambda i: (0, i, 0)),
            pl.BlockSpec((D, D), lambda i: (0, 0)),
        ],
        out_specs=pl.BlockSpec((ROWS_BLK, D), lambda i: (i, 0)),
        out_shape=jax.ShapeDtypeStruct((N_NODES, D), jnp.float32),
    )(partials, W)


def kernel(x, edge_index, W):
    src = edge_index[0]
    dst = edge_index[1]
    partials = _sc_scatter(x, src, dst)
    return _combine(partials, W)
